# baseline (speedup 1.0000x reference)
import functools

import jax
import jax.numpy as jnp
from jax import lax
from jax.experimental import pallas as pl
from jax.experimental.pallas import tpu as pltpu
from jax.experimental.pallas import tpu_sc as plsc

NC = 2
NS = 16
NW = NC * NS

SPLIT = 2
NBUF = 10
GPRE = 6


def _make_gather(batch: int, seq: int, dim: int):
  assert batch % (NW * SPLIT) == 0
  bpw = batch // NW
  cs = bpw // SPLIT
  nchunk = seq * SPLIT
  assert cs <= 128
  assert nchunk % NBUF == 0

  mesh = plsc.VectorSubcoreMesh(core_axis_name="c", subcore_axis_name="s")

  @functools.partial(
      pl.kernel,
      out_type=jax.ShapeDtypeStruct((seq, batch, dim), jnp.float32),
      mesh=mesh,
      scratch_types=[
          pltpu.VMEM((seq, bpw), jnp.int32),
          [pltpu.VMEM((cs, dim), jnp.float32) for _ in range(NBUF)],
          [pltpu.SemaphoreType.DMA for _ in range(NBUF)],
          [pltpu.SemaphoreType.DMA for _ in range(NBUF)],
      ],
  )
  def gather_kernel(table_hbm, idx_hbm, out_hbm, idx_v, bufs, in_sems, out_sems):
    wid = lax.axis_index("s") * NC + lax.axis_index("c")
    batch_base = wid * bpw

    pltpu.sync_copy(idx_hbm.at[:, wid], idx_v)

    def start_gather(c, b):
      l, half = c // SPLIT, c % SPLIT
      pltpu.async_copy(table_hbm.at[idx_v.at[l, pl.ds(half * cs, cs)]],
                       bufs[b], in_sems[b])

    def wait_sem(sems, b):
      pltpu.make_async_copy(table_hbm.at[pl.ds(0, cs)], bufs[b], sems[b]).wait()

    def start_out(c, b):
      l, half = c // SPLIT, c % SPLIT
      pltpu.async_copy(bufs[b],
                       out_hbm.at[l, pl.ds(batch_base + half * cs, cs)],
                       out_sems[b])

    for c in range(GPRE):
      start_gather(c, c % NBUF)

    @pl.loop(0, nchunk, step=NBUF)
    def _(g):
      for b in range(NBUF):
        c = g + b
        bn = (b + GPRE) % NBUF

        @pl.when(c + GPRE < nchunk)
        def _():
          @pl.when(c + GPRE >= NBUF)
          def _():
            wait_sem(out_sems, bn)

          start_gather(c + GPRE, bn)

        wait_sem(in_sems, b)
        start_out(c, b)

    for b in range(NBUF):
      wait_sem(out_sems, b)

  return gather_kernel


@jax.jit
def kernel(x, table):
  batch, seq = x.shape
  dim = table.shape[1]
  xt = x.astype(jnp.int32).T.reshape(seq, NW, batch // NW)
  out = _make_gather(batch, seq, dim)(table, xt)
  return out.transpose(1, 0, 2)
